# baseline (speedup 1.0000x reference)
import jax
import jax.numpy as jnp
from jax.experimental import pallas as pl
from jax.experimental.pallas import tpu as pltpu

_THRESHOLD = 0.3
_PENALTY_WEIGHT = 0.05

_B, _H, _W, _C = 8, 384, 384, 96
_BH = 32
_ROWS = _BH * _W
_NSPLIT = 4
_SUB = _ROWS // _NSPLIT
_CHUNKS_PER_B = _H // _BH
_NCHUNKS = _B * _CHUNKS_PER_B


def _stage1(xp_hbm, xt_hbm, rowp_out, rowt_out, zp_out, zt_out,
            bufp, buft, zp_acc, zt_acc, rowp, rowt, sem_in, sem_out):
    i = pl.program_id(0)
    h = i % _CHUNKS_PER_B
    b = i // _CHUNKS_PER_B
    slot = i % 2

    xpf = xp_hbm.reshape(_NCHUNKS, _ROWS, _C)
    xtf = xt_hbm.reshape(_NCHUNKS, _ROWS, _C)

    def copies(j, s):
        res = []
        for k in range(_NSPLIT):
            rows = pl.ds(k * _SUB, _SUB)
            res.append(pltpu.make_async_copy(
                xpf.at[j, rows, :], bufp.at[s, rows, :], sem_in.at[s, 0, k]))
            res.append(pltpu.make_async_copy(
                xtf.at[j, rows, :], buft.at[s, rows, :], sem_in.at[s, 1, k]))
        return res

    @pl.when(i == 0)
    def _():
        for c in copies(0, 0):
            c.start()

    @pl.when(i + 1 < _NCHUNKS)
    def _():
        for c in copies(i + 1, (i + 1) % 2):
            c.start()

    for c in copies(i, slot):
        c.wait()

    xp = bufp[slot].reshape(_BH, _W, _C)
    xt = buft[slot].reshape(_BH, _W, _C)
    rowp[pl.ds(i * _BH, _BH), :] = jnp.max(xp, axis=1)
    rowt[pl.ds(i * _BH, _BH), :] = jnp.max(xt, axis=1)
    zp = jnp.max(xp, axis=0)
    zt = jnp.max(xt, axis=0)

    @pl.when(h == 0)
    def _():
        zp_acc[...] = zp
        zt_acc[...] = zt

    @pl.when(h != 0)
    def _():
        zp_acc[...] = jnp.maximum(zp_acc[...], zp)
        zt_acc[...] = jnp.maximum(zt_acc[...], zt)

    @pl.when(h == _CHUNKS_PER_B - 1)
    def _():
        cp = pltpu.make_async_copy(zp_acc, zp_out.at[b], sem_out.at[0])
        ct = pltpu.make_async_copy(zt_acc, zt_out.at[b], sem_out.at[1])
        cp.start()
        ct.start()
        cp.wait()
        ct.wait()

    @pl.when(i == _NCHUNKS - 1)
    def _():
        cp = pltpu.make_async_copy(rowp, rowp_out, sem_out.at[0])
        ct = pltpu.make_async_copy(rowt, rowt_out, sem_out.at[1])
        cp.start()
        ct.start()
        cp.wait()
        ct.wait()


def _bounds(vals, thr, size):
    mask = vals > thr
    idx = jax.lax.broadcasted_iota(jnp.int32, vals.shape, 1)
    mn = jnp.min(jnp.where(mask, idx, size), axis=1, keepdims=True)
    mx = jnp.max(jnp.where(mask, idx, -1), axis=1, keepdims=True)
    empty = mn == size
    mn = jnp.where(empty, 0, mn)
    mx = jnp.where(empty, 1, mx)
    return mn.astype(jnp.float32), mx.astype(jnp.float32)


def _stage2(rowp_ref, rowt_ref, zp_ref, zt_ref, out_ref):
    rowp = jnp.max(rowp_ref[...].reshape(_B, _H, _C), axis=2)
    rowt = jnp.max(rowt_ref[...].reshape(_B, _H, _C), axis=2)
    colp = jnp.max(zp_ref[...], axis=2)
    colt = jnp.max(zt_ref[...], axis=2)
    p_y1, p_y2 = _bounds(rowp, _THRESHOLD, _H)
    p_x1, p_x2 = _bounds(colp, _THRESHOLD, _W)
    t_y1, t_y2 = _bounds(rowt, 0.5, _H)
    t_x1, t_x2 = _bounds(colt, 0.5, _W)

    pred_area = (p_y2 - p_y1 + 1.0) * (p_x2 - p_x1 + 1.0)
    true_area = (t_y2 - t_y1 + 1.0) * (t_x2 - t_x1 + 1.0)
    area_penalty = jnp.maximum(pred_area - true_area, 0.0) / (true_area + 1.0)
    dy = (p_y1 + p_y2 - t_y1 - t_y2) * 0.5
    dx = (p_x1 + p_x2 - t_x1 - t_x2) * 0.5
    center_offset = jnp.sqrt(dy * dy + dx * dx) / 20.0
    penalty = area_penalty + center_offset
    out_ref[...] = (_PENALTY_WEIGHT / _B) * jnp.sum(penalty, axis=0, keepdims=True)


def kernel(prediction_probs, expected_onehot):
    rowp, rowt, zp, zt = pl.pallas_call(
        _stage1,
        grid=(_NCHUNKS,),
        in_specs=[
            pl.BlockSpec(memory_space=pltpu.HBM),
            pl.BlockSpec(memory_space=pltpu.HBM),
        ],
        out_specs=[
            pl.BlockSpec(memory_space=pltpu.HBM),
            pl.BlockSpec(memory_space=pltpu.HBM),
            pl.BlockSpec(memory_space=pltpu.HBM),
            pl.BlockSpec(memory_space=pltpu.HBM),
        ],
        out_shape=[
            jax.ShapeDtypeStruct((_B * _H, _C), jnp.float32),
            jax.ShapeDtypeStruct((_B * _H, _C), jnp.float32),
            jax.ShapeDtypeStruct((_B, _W, _C), jnp.float32),
            jax.ShapeDtypeStruct((_B, _W, _C), jnp.float32),
        ],
        scratch_shapes=[
            pltpu.VMEM((2, _ROWS, _C), jnp.float32),
            pltpu.VMEM((2, _ROWS, _C), jnp.float32),
            pltpu.VMEM((_W, _C), jnp.float32),
            pltpu.VMEM((_W, _C), jnp.float32),
            pltpu.VMEM((_B * _H, _C), jnp.float32),
            pltpu.VMEM((_B * _H, _C), jnp.float32),
            pltpu.SemaphoreType.DMA((2, 2, _NSPLIT)),
            pltpu.SemaphoreType.DMA((2,)),
        ],
    )(prediction_probs, expected_onehot)

    out = pl.pallas_call(
        _stage2,
        out_shape=jax.ShapeDtypeStruct((1, 1), jnp.float32),
    )(rowp, rowt, zp, zt)
    return out[0, 0]


# speedup vs baseline: 4.7315x; 4.7315x over previous
import jax
import jax.numpy as jnp
from jax.experimental import pallas as pl
from jax.experimental.pallas import tpu as pltpu

_THRESHOLD = 0.3
_PENALTY_WEIGHT = 0.05

_B, _H, _W, _C = 8, 384, 384, 96
_BH = 32


def _stage1(xp_ref, xt_ref, rowp_ref, rowt_ref, zp_ref, zt_ref):
    h = pl.program_id(1)
    xp = xp_ref[0]
    xt = xt_ref[0]
    rowp_ref[0] = jnp.max(xp, axis=1)
    rowt_ref[0] = jnp.max(xt, axis=1)
    zp = jnp.max(xp, axis=0)
    zt = jnp.max(xt, axis=0)

    @pl.when(h == 0)
    def _():
        zp_ref[0] = zp
        zt_ref[0] = zt

    @pl.when(h != 0)
    def _():
        zp_ref[0] = jnp.maximum(zp_ref[0], zp)
        zt_ref[0] = jnp.maximum(zt_ref[0], zt)


def _bounds(vals, thr, size):
    mask = vals > thr
    idx = jax.lax.broadcasted_iota(jnp.int32, vals.shape, 1)
    mn = jnp.min(jnp.where(mask, idx, size), axis=1, keepdims=True)
    mx = jnp.max(jnp.where(mask, idx, -1), axis=1, keepdims=True)
    empty = mn == size
    mn = jnp.where(empty, 0, mn)
    mx = jnp.where(empty, 1, mx)
    return mn.astype(jnp.float32), mx.astype(jnp.float32)


def _stage2(rowp_ref, rowt_ref, zp_ref, zt_ref, out_ref):
    rowp = jnp.max(rowp_ref[...], axis=2)
    rowt = jnp.max(rowt_ref[...], axis=2)
    colp = jnp.max(zp_ref[...], axis=1)
    colt = jnp.max(zt_ref[...], axis=1)
    p_y1, p_y2 = _bounds(rowp, _THRESHOLD, _H)
    p_x1, p_x2 = _bounds(colp, _THRESHOLD, _W)
    t_y1, t_y2 = _bounds(rowt, 0.5, _H)
    t_x1, t_x2 = _bounds(colt, 0.5, _W)

    pred_area = (p_y2 - p_y1 + 1.0) * (p_x2 - p_x1 + 1.0)
    true_area = (t_y2 - t_y1 + 1.0) * (t_x2 - t_x1 + 1.0)
    area_penalty = jnp.maximum(pred_area - true_area, 0.0) / (true_area + 1.0)
    dy = (p_y1 + p_y2 - t_y1 - t_y2) * 0.5
    dx = (p_x1 + p_x2 - t_x1 - t_x2) * 0.5
    center_offset = jnp.sqrt(dy * dy + dx * dx) / 20.0
    penalty = area_penalty + center_offset
    out_ref[...] = (_PENALTY_WEIGHT / _B) * jnp.sum(penalty, axis=0, keepdims=True)


def kernel(prediction_probs, expected_onehot):
    xp = jnp.transpose(prediction_probs, (0, 1, 3, 2))
    xt = jnp.transpose(expected_onehot, (0, 1, 3, 2))
    rowp, rowt, zp, zt = pl.pallas_call(
        _stage1,
        grid=(_B, _H // _BH),
        in_specs=[
            pl.BlockSpec((1, _BH, _C, _W), lambda b, h: (b, h, 0, 0)),
            pl.BlockSpec((1, _BH, _C, _W), lambda b, h: (b, h, 0, 0)),
        ],
        out_specs=[
            pl.BlockSpec((1, _BH, _W), lambda b, h: (b, h, 0)),
            pl.BlockSpec((1, _BH, _W), lambda b, h: (b, h, 0)),
            pl.BlockSpec((1, _C, _W), lambda b, h: (b, 0, 0)),
            pl.BlockSpec((1, _C, _W), lambda b, h: (b, 0, 0)),
        ],
        out_shape=[
            jax.ShapeDtypeStruct((_B, _H, _W), jnp.float32),
            jax.ShapeDtypeStruct((_B, _H, _W), jnp.float32),
            jax.ShapeDtypeStruct((_B, _C, _W), jnp.float32),
            jax.ShapeDtypeStruct((_B, _C, _W), jnp.float32),
        ],
        compiler_params=pltpu.CompilerParams(
            dimension_semantics=("parallel", "arbitrary"),
        ),
    )(xp, xt)

    out = pl.pallas_call(
        _stage2,
        out_shape=jax.ShapeDtypeStruct((1, 1), jnp.float32),
    )(rowp, rowt, zp, zt)
    return out[0, 0]
